# Initial kernel scaffold; baseline (speedup 1.0000x reference)
import jax
import jax.numpy as jnp
from jax.experimental import pallas as pl


def kernel(x, position_ids, cos_cached, sin_cached):
    raise NotImplementedError("write your pallas kernel here")


# speedup vs baseline: 3.2115x; 3.2115x over previous
import functools

import jax
import jax.numpy as jnp
from jax import lax
from jax.experimental import pallas as pl
from jax.experimental.pallas import tpu as pltpu
from jax.experimental.pallas import tpu_sc as plsc

DIM = 64

_info = plsc.get_sparse_core_info()
_NC, _NS = _info.num_cores, _info.num_subcores
_NW = _NC * _NS

_CHUNK = 128


@functools.partial(jax.jit, static_argnums=())
def _gather_pallas(cos_cached, sin_cached, idx):
    n = idx.shape[0]
    b_per_w = n // _NW
    half = b_per_w // 2
    n_chunks = half // _CHUNK

    mesh = plsc.VectorSubcoreMesh(core_axis_name="c", subcore_axis_name="s")

    @functools.partial(
        pl.kernel,
        mesh=mesh,
        out_type=[
            jax.ShapeDtypeStruct((n, DIM), jnp.float32),
            jax.ShapeDtypeStruct((n, DIM), jnp.float32),
        ],
        scratch_types=[
            pltpu.VMEM((b_per_w,), jnp.int32),
            pltpu.VMEM((half, DIM), jnp.float32),
            pltpu.VMEM((half, DIM), jnp.float32),
            pltpu.SemaphoreType.DMA,
        ],
        compiler_params=pltpu.CompilerParams(use_tc_tiling_on_sc=False),
    )
    def k(cos_hbm, sin_hbm, idx_hbm, cos_out, sin_out, idx_v, cos_v, sin_v, sem):
        wid = lax.axis_index("s") * _NC + lax.axis_index("c")
        base = wid * b_per_w
        pltpu.sync_copy(idx_hbm.at[pl.ds(base, b_per_w)], idx_v)
        for h in range(2):
            for j in range(n_chunks):
                off = h * half + j * _CHUNK
                idx_c = idx_v.at[pl.ds(off, _CHUNK)]
                pltpu.async_copy(cos_hbm.at[idx_c], cos_v.at[pl.ds(j * _CHUNK, _CHUNK)], sem)
                pltpu.async_copy(sin_hbm.at[idx_c], sin_v.at[pl.ds(j * _CHUNK, _CHUNK)], sem)
            for j in range(n_chunks):
                pltpu.make_async_copy(cos_hbm.at[pl.ds(0, _CHUNK)], cos_v.at[pl.ds(j * _CHUNK, _CHUNK)], sem).wait()
                pltpu.make_async_copy(sin_hbm.at[pl.ds(0, _CHUNK)], sin_v.at[pl.ds(j * _CHUNK, _CHUNK)], sem).wait()
            pltpu.sync_copy(cos_v, cos_out.at[pl.ds(base + h * half, half)])
            pltpu.sync_copy(sin_v, sin_out.at[pl.ds(base + h * half, half)])

    return k(cos_cached, sin_cached, idx)


def kernel(x, position_ids, cos_cached, sin_cached):
    b, s = position_ids.shape
    idx = position_ids.reshape(-1).astype(jnp.int32)
    cos_flat, sin_flat = _gather_pallas(cos_cached, sin_cached, idx)
    cos = cos_flat.reshape(b, s, DIM).astype(x.dtype)
    sin = sin_flat.reshape(b, s, DIM).astype(x.dtype)
    return (cos, sin)
